# baseline (speedup 1.0000x reference)
import functools

import jax
import jax.numpy as jnp
from jax import lax
from jax.experimental import pallas as pl
from jax.experimental.pallas import tpu as pltpu
from jax.experimental.pallas import tpu_sc as plsc

B = 1024
L = 200
VOCAB_P = 400
DW = 32
DWP = 128
DP = 16
WIN = 3
D = DW + 2 * DP
TP = 208
NBLK = TP // 16
RSZ = 3 * TP
NC = 2
NS = 16
NW = NC * NS
ROWS_PER_W = B // NW


def _build_sc_call():
    mesh = plsc.VectorSubcoreMesh(core_axis_name="c", subcore_axis_name="s")

    @functools.partial(
        pl.kernel,
        mesh=mesh,
        compiler_params=pltpu.CompilerParams(use_tc_tiling_on_sc=False,
                                             needs_layout_passes=False),
        out_type=jax.ShapeDtypeStruct((B, L, WIN * D), jnp.float32),
        scratch_types=[
            pltpu.VMEM((ROWS_PER_W * RSZ,), jnp.int32),
            pltpu.VMEM((TP, DWP), jnp.float32),
            pltpu.VMEM((TP, DWP), jnp.float32),
            pltpu.VMEM((TP, 2 * DP), jnp.float32),
            pltpu.VMEM((TP, 2 * DP), jnp.float32),
            pltpu.VMEM((VOCAB_P, DP), jnp.float32),
            pltpu.SemaphoreType.DMA,
            pltpu.SemaphoreType.DMA,
        ],
    )
    def sc_kernel(idxs, we, wpe, out, ids, web0, web1, peb0, peb1, wpe_v,
                  gsem0, gsem1):
        wid = lax.axis_index("s") * NC + lax.axis_index("c")
        web = (web0, web1)
        peb = (peb0, peb1)
        gsem = (gsem0, gsem1)
        pltpu.sync_copy(wpe, wpe_v)
        pltpu.sync_copy(idxs.at[pl.ds(wid * (ROWS_PER_W * RSZ),
                                      ROWS_PER_W * RSZ)], ids)
        iota = lax.iota(jnp.int32, 16)

        def issue_gather(row, s):
            idx_ref = ids.at[pl.ds(row * RSZ, TP)]
            pltpu.async_copy(we.at[idx_ref], web[s], gsem[s])

        def fill_pe(row, s):
            def blk_body(blk, carry):
                t0 = blk * 16
                tvec = t0 + iota
                for tbl in range(2):
                    idv = ids[pl.ds(row * RSZ + TP + tbl * TP + t0, 16)]
                    for c in range(DP):
                        cvec = jnp.full((16,), c, jnp.int32)
                        val = plsc.load_gather(wpe_v, [idv, cvec])
                        plsc.store_scatter(
                            peb[s],
                            [tvec, jnp.full((16,), c + tbl * DP, jnp.int32)],
                            val)
                return carry

            lax.fori_loop(0, NBLK, blk_body, 0)

        def wait_gather(s):
            pltpu.make_async_copy(we.at[ids.at[pl.ds(0, TP)]], web[s],
                                  gsem[s]).wait()

        issue_gather(0, 0)
        fill_pe(0, 0)

        def body(k, carry):
            for s in range(2):
                it = 2 * k + s
                b = wid * ROWS_PER_W + it
                wait_gather(s)
                if s == 0:
                    issue_gather(it + 1, 1)
                    fill_pe(it + 1, 1)
                else:

                    @pl.when(k < ROWS_PER_W // 2 - 1)
                    def _():
                        issue_gather(it + 1, 0)
                        fill_pe(it + 1, 0)

                for j in range(WIN):
                    win = pl.ds(j, L)
                    col = j * D
                    pltpu.sync_copy(web[s].at[win, pl.ds(0, DW)],
                                    out.at[b, :, pl.ds(col, DW)])
                    pltpu.sync_copy(peb[s].at[win, :],
                                    out.at[b, :, pl.ds(col + DW, 2 * DP)])
            return carry

        lax.fori_loop(0, ROWS_PER_W // 2, body, 0)

    return sc_kernel


_SC_CALL = _build_sc_call()


def kernel(seq_inputs, e1_pos_inputs, e2_pos_inputs, we_table, wpe_table):
    b, l = seq_inputs.shape
    zero1 = jnp.zeros((b, 1), jnp.int32)
    pad6 = jnp.zeros((b, TP - l - 2), jnp.int32)
    seq_p = jnp.concatenate([zero1, seq_inputs, zero1, pad6], axis=1)
    e1_p = jnp.concatenate(
        [e1_pos_inputs[:, :1], e1_pos_inputs, e1_pos_inputs[:, -1:], pad6],
        axis=1)
    e2_p = jnp.concatenate(
        [e2_pos_inputs[:, :1], e2_pos_inputs, e2_pos_inputs[:, -1:], pad6],
        axis=1)
    idx_all = jnp.stack([seq_p, e1_p, e2_p], axis=1).reshape(-1)
    we_pad = jnp.pad(we_table, ((0, 0), (0, DWP - DW)))
    return _SC_CALL(idx_all, we_pad, wpe_table)


# speedup vs baseline: 1.5816x; 1.5816x over previous
import functools

import jax
import jax.numpy as jnp
from jax import lax
from jax.experimental import pallas as pl
from jax.experimental.pallas import tpu as pltpu
from jax.experimental.pallas import tpu_sc as plsc

B = 1024
L = 200
VOCAB_P = 400
DW = 32
DWP = 128
DP = 16
WIN = 3
D = DW + 2 * DP
TP = 208
NBLK = TP // 16
RSZ = 3 * TP
NC = 2
NS = 16
NW = NC * NS
ROWS_PER_W = B // NW


def _build_sc_call():
    mesh = plsc.VectorSubcoreMesh(core_axis_name="c", subcore_axis_name="s")

    @functools.partial(
        pl.kernel,
        mesh=mesh,
        compiler_params=pltpu.CompilerParams(use_tc_tiling_on_sc=False,
                                             needs_layout_passes=False),
        out_type=jax.ShapeDtypeStruct((B, L, WIN * D), jnp.float32),
        scratch_types=[
            pltpu.VMEM((ROWS_PER_W * RSZ,), jnp.int32),
            pltpu.VMEM((TP, DW), jnp.float32),
            pltpu.VMEM((TP, DW), jnp.float32),
            pltpu.VMEM((TP, 2 * DP), jnp.float32),
            pltpu.VMEM((TP, 2 * DP), jnp.float32),
            pltpu.VMEM((VOCAB_P, DP), jnp.float32),
            pltpu.SemaphoreType.DMA,
            pltpu.SemaphoreType.DMA,
        ],
    )
    def sc_kernel(idxs, we, wpe, out, ids, web0, web1, peb0, peb1, wpe_v,
                  gsem0, gsem1):
        wid = lax.axis_index("s") * NC + lax.axis_index("c")
        web = (web0, web1)
        peb = (peb0, peb1)
        gsem = (gsem0, gsem1)
        pltpu.sync_copy(wpe, wpe_v)
        pltpu.sync_copy(idxs.at[pl.ds(wid * (ROWS_PER_W * RSZ),
                                      ROWS_PER_W * RSZ)], ids)
        iota = lax.iota(jnp.int32, 16)

        def issue_gather(row, s):
            idx_ref = ids.at[pl.ds(row * RSZ, TP)]
            pltpu.async_copy(we.at[idx_ref], web[s], gsem[s])

        def fill_pe(row, s):
            def blk_body(blk, carry):
                t0 = blk * 16
                tvec = t0 + iota
                for tbl in range(2):
                    idv = ids[pl.ds(row * RSZ + TP + tbl * TP + t0, 16)]
                    for c in range(DP):
                        cvec = jnp.full((16,), c, jnp.int32)
                        val = plsc.load_gather(wpe_v, [idv, cvec])
                        plsc.store_scatter(
                            peb[s],
                            [tvec, jnp.full((16,), c + tbl * DP, jnp.int32)],
                            val)
                return carry

            lax.fori_loop(0, NBLK, blk_body, 0)

        def wait_gather(s):
            pltpu.make_async_copy(we.at[ids.at[pl.ds(0, TP)]], web[s],
                                  gsem[s]).wait()

        issue_gather(0, 0)
        fill_pe(0, 0)

        def body(k, carry):
            for s in range(2):
                it = 2 * k + s
                b = wid * ROWS_PER_W + it
                wait_gather(s)
                if s == 0:
                    issue_gather(it + 1, 1)
                    fill_pe(it + 1, 1)
                else:

                    @pl.when(k < ROWS_PER_W // 2 - 1)
                    def _():
                        issue_gather(it + 1, 0)
                        fill_pe(it + 1, 0)

                for j in range(WIN):
                    win = pl.ds(j, L)
                    col = j * D
                    pltpu.sync_copy(web[s].at[win, :],
                                    out.at[b, :, pl.ds(col, DW)])
                    pltpu.sync_copy(peb[s].at[win, :],
                                    out.at[b, :, pl.ds(col + DW, 2 * DP)])
            return carry

        lax.fori_loop(0, ROWS_PER_W // 2, body, 0)

    return sc_kernel


_SC_CALL = _build_sc_call()


def kernel(seq_inputs, e1_pos_inputs, e2_pos_inputs, we_table, wpe_table):
    b, l = seq_inputs.shape
    zero1 = jnp.zeros((b, 1), jnp.int32)
    pad6 = jnp.zeros((b, TP - l - 2), jnp.int32)
    seq_p = jnp.concatenate([zero1, seq_inputs, zero1, pad6], axis=1)
    e1_p = jnp.concatenate(
        [e1_pos_inputs[:, :1], e1_pos_inputs, e1_pos_inputs[:, -1:], pad6],
        axis=1)
    e2_p = jnp.concatenate(
        [e2_pos_inputs[:, :1], e2_pos_inputs, e2_pos_inputs[:, -1:], pad6],
        axis=1)
    idx_all = jnp.stack([seq_p, e1_p, e2_p], axis=1).reshape(-1)
    return _SC_CALL(idx_all, we_table, wpe_table)


# speedup vs baseline: 1.6227x; 1.0260x over previous
import functools

import jax
import jax.numpy as jnp
from jax import lax
from jax.experimental import pallas as pl
from jax.experimental.pallas import tpu as pltpu
from jax.experimental.pallas import tpu_sc as plsc

B = 1024
L = 200
VOCAB_P = 400
DW = 32
DP = 16
WIN = 3
D = DW + 2 * DP
TP = 208
NBLK = TP // 16
RSZ = 3 * TP
NB = 4
SSZ = NB * RSZ
NC = 2
NS = 16
NW = NC * NS
ROWS_PER_W = B // NW
STEPS = ROWS_PER_W // NB


def _build_sc_call():
    mesh = plsc.VectorSubcoreMesh(core_axis_name="c", subcore_axis_name="s")

    @functools.partial(
        pl.kernel,
        mesh=mesh,
        compiler_params=pltpu.CompilerParams(use_tc_tiling_on_sc=False,
                                             needs_layout_passes=False),
        out_type=jax.ShapeDtypeStruct((B, L, WIN * D), jnp.float32),
        scratch_types=[
            pltpu.VMEM((SSZ,), jnp.int32),
            pltpu.VMEM((SSZ,), jnp.int32),
            pltpu.VMEM((NB, TP, DW), jnp.float32),
            pltpu.VMEM((NB, TP, DW), jnp.float32),
            pltpu.VMEM((NB, TP, 2 * DP), jnp.float32),
            pltpu.VMEM((NB, TP, 2 * DP), jnp.float32),
            pltpu.VMEM((VOCAB_P, DP), jnp.float32),
            pltpu.SemaphoreType.DMA,
            pltpu.SemaphoreType.DMA,
            pltpu.SemaphoreType.DMA,
            pltpu.SemaphoreType.DMA,
        ],
    )
    def sc_kernel(idxs, we, wpe, out, ids0, ids1, web0, web1, peb0, peb1,
                  wpe_v, gsem0, gsem1, wsem0, wsem1):
        wid = lax.axis_index("s") * NC + lax.axis_index("c")
        ids = (ids0, ids1)
        web = (web0, web1)
        peb = (peb0, peb1)
        gsem = (gsem0, gsem1)
        wsem = (wsem0, wsem1)
        pltpu.sync_copy(wpe, wpe_v)
        iota = lax.iota(jnp.int32, 16)

        def start_step(i, s):
            pltpu.sync_copy(
                idxs.at[pl.ds((wid * STEPS + i) * SSZ, SSZ)], ids[s])
            for r in range(NB):
                pltpu.async_copy(we.at[ids[s].at[pl.ds(r * RSZ, TP)]],
                                 web[s].at[r], gsem[s])

            def blk_body(q, carry):
                r = q // NBLK
                t0 = (q % NBLK) * 16
                rvec = jnp.full((16,), r, jnp.int32)
                tvec = t0 + iota
                for tbl in range(2):
                    idv = ids[s][pl.ds(r * RSZ + TP + tbl * TP + t0, 16)]
                    for c in range(DP):
                        cvec = jnp.full((16,), c, jnp.int32)
                        val = plsc.load_gather(wpe_v, [idv, cvec])
                        plsc.store_scatter(
                            peb[s],
                            [rvec, tvec,
                             jnp.full((16,), c + tbl * DP, jnp.int32)], val)
                return carry

            lax.fori_loop(0, NB * NBLK, blk_body, 0)

        def wait_gathers(s):
            for r in range(NB):
                pltpu.make_async_copy(we.at[ids[s].at[pl.ds(r * RSZ, TP)]],
                                      web[s].at[r], gsem[s]).wait()

        def write_refs(i, s):
            rows = pl.ds(wid * ROWS_PER_W + i * NB, NB)
            for j in range(WIN):
                win = pl.ds(j, L)
                col = j * D
                yield web[s].at[:, win, :], out.at[rows, :, pl.ds(col, DW)]
                yield (peb[s].at[:, win, :],
                       out.at[rows, :, pl.ds(col + DW, 2 * DP)])

        def issue_writes(i, s):
            for src, dst in write_refs(i, s):
                pltpu.async_copy(src, dst, wsem[s])

        def drain_writes(i, s):
            for src, dst in write_refs(i, s):
                pltpu.make_async_copy(src, dst, wsem[s]).wait()

        start_step(0, 0)

        def body(k, carry):
            for s in range(2):
                i = 2 * k + s
                wait_gathers(s)
                issue_writes(i, s)

                @pl.when(i >= 1)
                def _():
                    drain_writes(i, 1 - s)

                @pl.when(i < STEPS - 1)
                def _():
                    start_step(i + 1, 1 - s)

            return carry

        lax.fori_loop(0, STEPS // 2, body, 0)
        drain_writes(STEPS - 1, 1)

    return sc_kernel


_SC_CALL = _build_sc_call()


def kernel(seq_inputs, e1_pos_inputs, e2_pos_inputs, we_table, wpe_table):
    b, l = seq_inputs.shape
    zero1 = jnp.zeros((b, 1), jnp.int32)
    pad6 = jnp.zeros((b, TP - l - 2), jnp.int32)
    seq_p = jnp.concatenate([zero1, seq_inputs, zero1, pad6], axis=1)
    e1_p = jnp.concatenate(
        [e1_pos_inputs[:, :1], e1_pos_inputs, e1_pos_inputs[:, -1:], pad6],
        axis=1)
    e2_p = jnp.concatenate(
        [e2_pos_inputs[:, :1], e2_pos_inputs, e2_pos_inputs[:, -1:], pad6],
        axis=1)
    idx_all = jnp.stack([seq_p, e1_p, e2_p], axis=1).reshape(-1)
    return _SC_CALL(idx_all, we_table, wpe_table)
